# baseline (speedup 1.0000x reference)
import jax
import jax.numpy as jnp
from jax.experimental import pallas as pl
from jax.experimental.pallas import tpu as pltpu

_BM = 200


def _gcn_block_kernel(x_ref, adj_ref, w_ref, b_ref, out_ref, support_ref):
    i = pl.program_id(0)

    @pl.when(i == 0)
    def _():
        support_ref[...] = jnp.dot(
            x_ref[...], w_ref[...], preferred_element_type=jnp.float32
        )

    out = jnp.dot(
        adj_ref[...], support_ref[...], preferred_element_type=jnp.float32
    )
    h = jnp.maximum(out + b_ref[...], 0.0)
    m = jnp.max(h, axis=1, keepdims=True)
    lse = m + jnp.log(jnp.sum(jnp.exp(h - m), axis=1, keepdims=True))
    out_ref[pl.ds(i * _BM, _BM), :] = h - lse


def kernel(x, adj, W, b):
    n, feat = x.shape
    nclass = W.shape[1]
    b2 = b.reshape(1, nclass)
    return pl.pallas_call(
        _gcn_block_kernel,
        grid=(n // _BM,),
        in_specs=[
            pl.BlockSpec((n, feat), lambda i: (0, 0)),
            pl.BlockSpec((_BM, n), lambda i: (i, 0)),
            pl.BlockSpec((feat, nclass), lambda i: (0, 0)),
            pl.BlockSpec((1, nclass), lambda i: (0, 0)),
        ],
        out_specs=pl.BlockSpec((n, nclass), lambda i: (0, 0)),
        out_shape=jax.ShapeDtypeStruct((n, nclass), jnp.float32),
        scratch_shapes=[pltpu.VMEM((n, nclass), jnp.float32)],
        compiler_params=pltpu.CompilerParams(
            vmem_limit_bytes=64 * 1024 * 1024,
        ),
    )(x, adj, W, b2)


# speedup vs baseline: 1.0182x; 1.0182x over previous
import jax
import jax.numpy as jnp
from jax.experimental import pallas as pl
from jax.experimental.pallas import tpu as pltpu

_BM = 200
_NBUF = 4


def _outer_kernel(x_ref, adj_ref, w_ref, b_ref, out_ref, support_ref):
    support_ref[...] = jnp.dot(
        x_ref[...], w_ref[...], preferred_element_type=jnp.float32
    )
    n = adj_ref.shape[0]

    def body(adj_blk_ref, out_blk_ref):
        out = jnp.dot(
            adj_blk_ref[...], support_ref[...],
            preferred_element_type=jnp.float32,
        )
        h = jnp.maximum(out + b_ref[...], 0.0)
        m = jnp.max(h, axis=1, keepdims=True)
        lse = m + jnp.log(jnp.sum(jnp.exp(h - m), axis=1, keepdims=True))
        out_blk_ref[...] = h - lse

    pipeline = pltpu.emit_pipeline(
        body,
        grid=(n // _BM,),
        in_specs=[
            pl.BlockSpec((_BM, n), lambda i: (i, 0),
                         pipeline_mode=pl.Buffered(buffer_count=_NBUF)),
        ],
        out_specs=[
            pl.BlockSpec((_BM, out_ref.shape[1]), lambda i: (i, 0)),
        ],
    )
    pipeline(adj_ref, out_ref)


def kernel(x, adj, W, b):
    n, feat = x.shape
    nclass = W.shape[1]
    b2 = b.reshape(1, nclass)
    return pl.pallas_call(
        _outer_kernel,
        in_specs=[
            pl.BlockSpec(memory_space=pltpu.VMEM),
            pl.BlockSpec(memory_space=pl.ANY),
            pl.BlockSpec(memory_space=pltpu.VMEM),
            pl.BlockSpec(memory_space=pltpu.VMEM),
        ],
        out_specs=pl.BlockSpec(memory_space=pl.ANY),
        out_shape=jax.ShapeDtypeStruct((n, nclass), jnp.float32),
        scratch_shapes=[pltpu.VMEM((n, nclass), jnp.float32)],
        compiler_params=pltpu.CompilerParams(
            vmem_limit_bytes=64 * 1024 * 1024,
        ),
    )(x, adj, W, b2)
